# baseline (speedup 1.0000x reference)
import functools

import jax
import jax.numpy as jnp
import numpy as np
from jax import lax
from jax.experimental import pallas as pl
from jax.experimental.pallas import tpu as pltpu
from jax.experimental.pallas import tpu_sc as plsc

_NC = 2
_NS = 16
_Q = 64
_BR = 4


def _build_plan(seq):
    total = seq * (seq + 1) // 2
    starts = [k * seq - (k * (k - 1)) // 2 for k in range(seq)]
    idx_np = np.concatenate(
        [np.arange(n, dtype=np.int32) for n in range(seq, 0, -1)])

    cls_items = {}
    covered = set()
    for k in range(seq):
        s, L = starts[k], seq - k
        h = (8 - s % 8) % 8
        m = ((s + L) // 8) * 8 - s
        j = 0
        while _Q * j + h < m:
            lo, hi = _Q * j + h, min(m, _Q * (j + 1) + h)
            cls_items.setdefault((j, h), []).append((hi - lo, s + lo))
            covered.update(range(s + lo, s + hi, 8))
            j += 1
    boundary = sorted(set(range(0, total, 8)) - covered)

    loads = {c: sum(L for L, _ in v) for c, v in cls_items.items()}
    order = sorted(loads, key=lambda c: -loads[c])
    npairs = len(order) // 2
    pairs = [(order[i], order[len(order) - 1 - i]) for i in range(npairs)]
    wload = [loads[a] + loads[b] for a, b in pairs]
    bassign = [[] for _ in range(npairs)]
    for blk in boundary:
        i = wload.index(min(wload))
        bassign[i].append(blk)
        wload[i] += 16

    idx_rows, bulk_items = [], []
    for P, (c1, c2) in enumerate(pairs):
        row = []
        items = []
        for slot, (j, h) in enumerate((c1, c2)):
            row.extend(range(_Q * j + h, _Q * j + h + _Q))
            items.extend((L, dst, slot * _Q, _Q * j if h == 0 else -1)
                         for L, dst in cls_items[(j, h)])
        for blk in bassign[P]:
            row.extend(int(v) for v in idx_np[blk:blk + 8])
        row.extend([0] * (4 * _Q - len(row)))
        idx_rows.append(row)
        bulk_items.append(items)
    return total, np.asarray(idx_rows, np.int32).reshape(-1), bulk_items, bassign


def kernel(inputs, dist_embedding):
    batch, seq = inputs.shape[0], inputs.shape[1]
    emb = dist_embedding.shape[1]
    assert batch == 2 and seq == 256
    total, idx_flat, bulk_items, bassign = _build_plan(seq)
    nrows = batch * total
    nslots = 4 * _Q

    mesh = plsc.VectorSubcoreMesh(core_axis_name="c", subcore_axis_name="s")

    @functools.partial(
        pl.kernel,
        mesh=mesh,
        out_type=jax.ShapeDtypeStruct((nrows, emb), jnp.float32),
        scratch_types=[
            pltpu.VMEM((nslots,), jnp.int32),
            pltpu.VMEM((2 * _Q, emb), jnp.float32),
            pltpu.VMEM((8 * _BR, emb), jnp.float32),
            pltpu.SemaphoreType.DMA,
            pltpu.SemaphoreType.DMA,
            pltpu.SemaphoreType.DMA,
            pltpu.SemaphoreType.DMA,
        ],
    )
    def _copy_kernel(table_hbm, idx_hbm, out_hbm, idx_v, tbuf, bbuf,
                     sem_stage, sem_bulk, sem_bg, sem_bs):
        wid = lax.axis_index("c") * _NS + lax.axis_index("s")
        pair = wid // batch
        b_off = (wid % batch) * total

        pltpu.sync_copy(idx_hbm.at[pl.ds(pair * nslots, nslots)], idx_v)
        pltpu.async_copy(
            table_hbm.at[idx_v.at[pl.ds(0, 2 * _Q)]], tbuf, sem_stage).wait()

        for P, (items, blocks) in enumerate(zip(bulk_items, bassign)):
            @pl.when(pair == P)
            def _(items=items, blocks=blocks):
                bulk = [
                    pltpu.async_copy(
                        tbuf.at[pl.ds(soff, L)] if hbm_lo < 0
                        else table_hbm.at[pl.ds(hbm_lo, L)],
                        out_hbm.at[pl.ds(b_off + dst, L)],
                        sem_bulk,
                    )
                    for L, dst, soff, hbm_lo in items
                ]
                for r0 in range(0, len(blocks), _BR):
                    blks = blocks[r0:r0 + _BR]
                    n = 8 * len(blks)
                    pltpu.async_copy(
                        table_hbm.at[idx_v.at[pl.ds(2 * _Q + 8 * r0, n)]],
                        bbuf.at[pl.ds(0, n)], sem_bg).wait()
                    scat = [
                        pltpu.async_copy(
                            bbuf.at[pl.ds(8 * q, 8)],
                            out_hbm.at[pl.ds(b_off + dst, 8)],
                            sem_bs,
                        )
                        for q, dst in enumerate(blks)
                    ]
                    for c in scat:
                        c.wait()
                for c in bulk:
                    c.wait()

    out = _copy_kernel(dist_embedding, jnp.asarray(idx_flat))
    return out.reshape(batch, total, emb)
